# baseline (speedup 1.0000x reference)
import jax
import jax.numpy as jnp
from jax.experimental import pallas as pl
from jax.experimental.pallas import tpu as pltpu

BLOCK_N = 512


def _assign_kernel(x_ref, cth_ref, ctl_ref, ct_ref, out_ref, cn2_ref):
    @pl.when(pl.program_id(0) == 0)
    def _():
        ct = ct_ref[...]
        cn2_ref[...] = jnp.sum(ct * ct, axis=0, keepdims=True)

    x_blk = x_ref[...]
    bits = jax.lax.bitcast_convert_type(x_blk, jnp.int32)
    xh_f32 = jax.lax.bitcast_convert_type(
        (bits + jnp.int32(0x8000)) & jnp.int32(-65536), jnp.float32)
    xh = xh_f32.astype(jnp.bfloat16)
    xl = (x_blk - xh_f32).astype(jnp.bfloat16)
    cth = cth_ref[...]
    ctl = ctl_ref[...]
    dot = lambda a, b: jnp.dot(a, b, preferred_element_type=jnp.float32)
    r2 = cn2_ref[...] + (dot(xh, cth) + (dot(xh, ctl) + dot(xl, cth)))
    xn2 = jnp.sum(x_blk * x_blk, axis=1, keepdims=True)
    d2 = jnp.maximum(xn2 + r2, 0.0)
    out_ref[...] = jnp.argmin(jnp.sqrt(d2), axis=1).astype(jnp.int32)


def _split_hi_lo(a):
    hi_f32 = jax.lax.reduce_precision(a, exponent_bits=8, mantissa_bits=7)
    return hi_f32.astype(jnp.bfloat16), (a - hi_f32).astype(jnp.bfloat16)


def kernel(x, centroids):
    n, d = x.shape
    k = centroids.shape[0]
    ct = centroids.T
    cth, ctl = _split_hi_lo(ct)
    cth = -2.0 * cth
    ctl = -2.0 * ctl
    grid = (n // BLOCK_N,)
    assignments = pl.pallas_call(
        _assign_kernel,
        grid=grid,
        in_specs=[
            pl.BlockSpec((BLOCK_N, d), lambda i: (i, 0)),
            pl.BlockSpec((d, k), lambda i: (0, 0)),
            pl.BlockSpec((d, k), lambda i: (0, 0)),
            pl.BlockSpec((d, k), lambda i: (0, 0)),
        ],
        out_specs=pl.BlockSpec((BLOCK_N,), lambda i: (i,)),
        out_shape=jax.ShapeDtypeStruct((n,), jnp.int32),
        scratch_shapes=[pltpu.VMEM((1, k), jnp.float32)],
    )(x, cth, ctl, ct)
    return (centroids[None, :, :], assignments)
